# Initial kernel scaffold; baseline (speedup 1.0000x reference)
import jax
import jax.numpy as jnp
from jax.experimental import pallas as pl


def kernel(x, features, edge_index, W, b):
    raise NotImplementedError("write your pallas kernel here")


# speedup vs baseline: 19.0340x; 19.0340x over previous
import functools

import jax
import jax.numpy as jnp
from jax import lax
from jax.experimental import pallas as pl
from jax.experimental.pallas import tpu as pltpu
from jax.experimental.pallas import tpu_sc as plsc

NC, NS = 2, 16
NW = NC * NS
L = 16
CH = 128

N = 10000
NPAD = 10240
E = 320000
EPW = NPAD
NCHUNK = EPW // CH
EPAD = NW * EPW
D = 128
H = 64
B = 4096
BPT = B // NS
ROWS_PT = NPAD // NS
RBLK = 512


def _fill1d(ref, val, n):
    def st(i, _):
        ref[pl.ds(i * L, L)] = jnp.full((L,), val, jnp.float32)
        return 0
    lax.fori_loop(0, n // L, st, 0)


def _deg_body(cols_hbm, deg0_hbm, deg1_hbm, idx_v, ones_v, zbuf_v, deg_sh):
    c = lax.axis_index("c")
    sid = lax.axis_index("s")
    wid = c * NS + sid
    _fill1d(ones_v, 1.0, CH)
    _fill1d(zbuf_v, 0.0, CH)

    def zs(t, _):
        pltpu.sync_copy(zbuf_v, deg_sh.at[pl.ds(sid * ROWS_PT + t * CH, CH)])
        return 0
    lax.fori_loop(0, ROWS_PT // CH, zs, 0)
    pltpu.sync_copy(cols_hbm.at[wid], idx_v)
    plsc.subcore_barrier()

    def scat(j, _):
        pltpu.sync_copy(ones_v, deg_sh.at[idx_v.at[j]], add=True)
        return 0
    lax.fori_loop(0, NCHUNK, scat, 0)
    plsc.subcore_barrier()

    sl = pl.ds(sid * ROWS_PT, ROWS_PT)

    @pl.when(c == 0)
    def _():
        pltpu.sync_copy(deg_sh.at[sl], deg0_hbm.at[sl])

    @pl.when(c == 1)
    def _():
        pltpu.sync_copy(deg_sh.at[sl], deg1_hbm.at[sl])


def _scat_body(rows_hbm, cols_hbm, xr_hbm, m_hbm, deg0_hbm, deg1_hbm,
               g0_hbm, g1_hbm, mx_hbm, d0x_hbm, d1x_hbm,
               ridx_v, cidx_v, x_v, buf_v, gbuf_v, dbuf_v, s_sh):
    c = lax.axis_index("c")
    sid = lax.axis_index("s")
    wid = c * NS + sid

    def zrow(i, _):
        def zc(k, _):
            buf_v[i, pl.ds(k * L, L)] = jnp.zeros((L,), jnp.float32)
            return 0
        lax.fori_loop(0, H // L, zc, 0)
        return 0
    lax.fori_loop(0, CH, zrow, 0)

    def zs(t, _):
        pltpu.sync_copy(buf_v, s_sh.at[pl.ds(sid * ROWS_PT + t * CH, CH)])
        return 0
    lax.fori_loop(0, ROWS_PT // CH, zs, 0)

    pltpu.sync_copy(rows_hbm.at[wid], ridx_v)
    pltpu.sync_copy(cols_hbm.at[wid], cidx_v)
    pltpu.sync_copy(xr_hbm.at[sid], x_v)
    plsc.subcore_barrier()

    def ed(j, _):
        pltpu.sync_copy(m_hbm.at[ridx_v.at[j]], buf_v)
        pltpu.sync_copy(buf_v, s_sh.at[cidx_v.at[j]], add=True)
        return 0
    lax.fori_loop(0, NCHUNK, ed, 0)
    plsc.subcore_barrier()

    def gk(k, _):
        idx = x_v.at[k]
        osl = pl.ds(sid * BPT + k * CH, CH)
        pltpu.sync_copy(s_sh.at[idx], gbuf_v)

        @pl.when(c == 0)
        def _():
            pltpu.sync_copy(gbuf_v, g0_hbm.at[osl])
            pltpu.sync_copy(m_hbm.at[idx], buf_v)
            pltpu.sync_copy(buf_v, mx_hbm.at[osl])

        @pl.when(c == 1)
        def _():
            pltpu.sync_copy(gbuf_v, g1_hbm.at[osl])
            pltpu.sync_copy(deg0_hbm.at[idx], dbuf_v)
            pltpu.sync_copy(dbuf_v, d0x_hbm.at[osl])
            pltpu.sync_copy(deg1_hbm.at[idx], dbuf_v)
            pltpu.sync_copy(dbuf_v, d1x_hbm.at[osl])
        return 0
    lax.fori_loop(0, BPT // CH, gk, 0)


def _mm_body(f_ref, w_ref, d0_ref, d1_ref, m_ref):
    dinv = lax.rsqrt(1.0 + d0_ref[...] + d1_ref[...])
    h = jnp.dot(f_ref[...], w_ref[...], preferred_element_type=jnp.float32)
    m_ref[...] = dinv * h


def _fin_body(g0_ref, g1_ref, mx_ref, d0_ref, d1_ref, b_ref, o_ref):
    scale = lax.rsqrt(1.0 + d0_ref[...] + d1_ref[...])
    o_ref[...] = scale * (g0_ref[...] + g1_ref[...] + mx_ref[...]) + b_ref[...]


def kernel(x, features, edge_index, W, b):
    row = edge_index[0]
    col = edge_index[1]
    pad = EPAD - E
    rows_p = jnp.concatenate([row, jnp.zeros((pad,), row.dtype)]).reshape(NW, NCHUNK, CH)
    cols_p = jnp.concatenate([col, jnp.full((pad,), N, col.dtype)]).reshape(NW, NCHUNK, CH)
    xr = x.reshape(NS, BPT // CH, CH)
    feats_p = jnp.pad(features, ((0, NPAD - N), (0, 0)))

    mesh = plsc.VectorSubcoreMesh(core_axis_name="c", subcore_axis_name="s",
                                  num_cores=NC, num_subcores=NS)
    sc_params = pltpu.CompilerParams(use_tc_tiling_on_sc=False)

    deg_call = pl.kernel(
        _deg_body,
        out_type=[jax.ShapeDtypeStruct((NPAD,), jnp.float32),
                  jax.ShapeDtypeStruct((NPAD,), jnp.float32)],
        mesh=mesh,
        scratch_types=[
            pltpu.VMEM((NCHUNK, CH), jnp.int32),
            pltpu.VMEM((CH,), jnp.float32),
            pltpu.VMEM((CH,), jnp.float32),
            pltpu.VMEM_SHARED((NPAD,), jnp.float32),
        ],
        compiler_params=sc_params,
    )
    deg0, deg1 = deg_call(cols_p)

    m = pl.pallas_call(
        _mm_body,
        grid=(NPAD // RBLK,),
        in_specs=[
            pl.BlockSpec((RBLK, D), lambda i: (i, 0)),
            pl.BlockSpec((D, H), lambda i: (0, 0)),
            pl.BlockSpec((RBLK, 1), lambda i: (i, 0)),
            pl.BlockSpec((RBLK, 1), lambda i: (i, 0)),
        ],
        out_specs=pl.BlockSpec((RBLK, H), lambda i: (i, 0)),
        out_shape=jax.ShapeDtypeStruct((NPAD, H), jnp.float32),
    )(feats_p, W, deg0.reshape(NPAD, 1), deg1.reshape(NPAD, 1))

    scat_call = pl.kernel(
        _scat_body,
        out_type=[jax.ShapeDtypeStruct((B, H), jnp.float32),
                  jax.ShapeDtypeStruct((B, H), jnp.float32),
                  jax.ShapeDtypeStruct((B, H), jnp.float32),
                  jax.ShapeDtypeStruct((B,), jnp.float32),
                  jax.ShapeDtypeStruct((B,), jnp.float32)],
        mesh=mesh,
        scratch_types=[
            pltpu.VMEM((NCHUNK, CH), jnp.int32),
            pltpu.VMEM((NCHUNK, CH), jnp.int32),
            pltpu.VMEM((BPT // CH, CH), jnp.int32),
            pltpu.VMEM((CH, H), jnp.float32),
            pltpu.VMEM((CH, H), jnp.float32),
            pltpu.VMEM((CH,), jnp.float32),
            pltpu.VMEM_SHARED((NPAD, H), jnp.float32),
        ],
        compiler_params=sc_params,
    )
    g0, g1, mx, d0x, d1x = scat_call(rows_p, cols_p, xr, m, deg0, deg1)

    out = pl.pallas_call(
        _fin_body,
        out_shape=jax.ShapeDtypeStruct((B, H), jnp.float32),
    )(g0, g1, mx, d0x.reshape(B, 1), d1x.reshape(B, 1), b.reshape(1, H))
    return out


# speedup vs baseline: 21.7676x; 1.1436x over previous
import functools

import jax
import jax.numpy as jnp
from jax import lax
from jax.experimental import pallas as pl
from jax.experimental.pallas import tpu as pltpu
from jax.experimental.pallas import tpu_sc as plsc

NC, NS = 2, 16
NW = NC * NS
L = 16
CH = 128

N = 10000
NPAD = 10240
E = 320000
EPW = NPAD
NCHUNK = EPW // CH
EPAD = NW * EPW
D = 128
H = 64
B = 4096
BPT = B // NS
ROWS_PT = NPAD // NS
RBLK = 512


def _fill1d(ref, val, n):
    def st(i, _):
        ref[pl.ds(i * L, L)] = jnp.full((L,), val, jnp.float32)
        return 0
    lax.fori_loop(0, n // L, st, 0)


def _deg_body(cols_hbm, deg0_hbm, deg1_hbm, idx_v, ones_v, zbuf_v, deg_sh):
    c = lax.axis_index("c")
    sid = lax.axis_index("s")
    wid = c * NS + sid
    _fill1d(ones_v, 1.0, CH)
    _fill1d(zbuf_v, 0.0, CH)

    def zs(t, _):
        pltpu.sync_copy(zbuf_v, deg_sh.at[pl.ds(sid * ROWS_PT + t * CH, CH)])
        return 0
    lax.fori_loop(0, ROWS_PT // CH, zs, 0)
    pltpu.sync_copy(cols_hbm.at[wid], idx_v)
    plsc.subcore_barrier()

    def scat(j, _):
        pltpu.sync_copy(ones_v, deg_sh.at[idx_v.at[j]], add=True)
        return 0
    lax.fori_loop(0, NCHUNK, scat, 0)
    plsc.subcore_barrier()

    sl = pl.ds(sid * ROWS_PT, ROWS_PT)

    @pl.when(c == 0)
    def _():
        pltpu.sync_copy(deg_sh.at[sl], deg0_hbm.at[sl])

    @pl.when(c == 1)
    def _():
        pltpu.sync_copy(deg_sh.at[sl], deg1_hbm.at[sl])


def _scat_body(rows_hbm, cols_hbm, xr_hbm, m_hbm, deg0_hbm, deg1_hbm,
               g0_hbm, g1_hbm, mx_hbm, d0x_hbm, d1x_hbm,
               ridx_v, cidx_v, x_v, buf_v, gbuf_v, dbuf_v, s_sh, semg, sems):
    c = lax.axis_index("c")
    sid = lax.axis_index("s")
    wid = c * NS + sid

    def zrow(i, _):
        def zc(k, _):
            buf_v[0, i, pl.ds(k * L, L)] = jnp.zeros((L,), jnp.float32)
            return 0
        lax.fori_loop(0, H // L, zc, 0)
        return 0
    lax.fori_loop(0, CH, zrow, 0)

    def zs(t, _):
        pltpu.sync_copy(buf_v.at[0], s_sh.at[pl.ds(sid * ROWS_PT + t * CH, CH)])
        return 0
    lax.fori_loop(0, ROWS_PT // CH, zs, 0)

    pltpu.sync_copy(rows_hbm.at[wid], ridx_v)
    pltpu.sync_copy(cols_hbm.at[wid], cidx_v)
    pltpu.sync_copy(xr_hbm.at[sid], x_v)
    plsc.subcore_barrier()

    pltpu.async_copy(m_hbm.at[ridx_v.at[0]], buf_v.at[0], semg)
    pltpu.async_copy(m_hbm.at[ridx_v.at[1]], buf_v.at[1], semg)

    def ed(j, _):
        @pl.when(j >= 2)
        def _():
            pltpu.make_async_copy(buf_v.at[0], s_sh.at[cidx_v.at[0]], sems).wait()

        @pl.when(j + 2 < NCHUNK)
        def _():
            p2 = lax.rem(j + 2, 4)
            pltpu.async_copy(m_hbm.at[ridx_v.at[j + 2]], buf_v.at[p2], semg)
        p = lax.rem(j, 4)
        pltpu.make_async_copy(m_hbm.at[ridx_v.at[0]], buf_v.at[p], semg).wait()
        pltpu.async_copy(buf_v.at[p], s_sh.at[cidx_v.at[j]], sems, add=True)
        return 0
    lax.fori_loop(0, NCHUNK, ed, 0)
    pltpu.make_async_copy(buf_v.at[0], s_sh.at[cidx_v.at[0]], sems).wait()
    pltpu.make_async_copy(buf_v.at[1], s_sh.at[cidx_v.at[1]], sems).wait()
    plsc.subcore_barrier()

    def gk(k, _):
        idx = x_v.at[k]
        osl = pl.ds(sid * BPT + k * CH, CH)
        pltpu.sync_copy(s_sh.at[idx], gbuf_v)

        @pl.when(c == 0)
        def _():
            pltpu.sync_copy(gbuf_v, g0_hbm.at[osl])
            pltpu.sync_copy(m_hbm.at[idx], buf_v.at[0])
            pltpu.sync_copy(buf_v.at[0], mx_hbm.at[osl])

        @pl.when(c == 1)
        def _():
            pltpu.sync_copy(gbuf_v, g1_hbm.at[osl])
            pltpu.sync_copy(deg0_hbm.at[idx], dbuf_v)
            pltpu.sync_copy(dbuf_v, d0x_hbm.at[osl])
            pltpu.sync_copy(deg1_hbm.at[idx], dbuf_v)
            pltpu.sync_copy(dbuf_v, d1x_hbm.at[osl])
        return 0
    lax.fori_loop(0, BPT // CH, gk, 0)


def _mm_body(f_ref, w_ref, d0_ref, d1_ref, m_ref):
    dinv = lax.rsqrt(1.0 + d0_ref[...] + d1_ref[...])
    h = jnp.dot(f_ref[...], w_ref[...], preferred_element_type=jnp.float32)
    m_ref[...] = dinv * h


def _fin_body(g0_ref, g1_ref, mx_ref, d0_ref, d1_ref, b_ref, o_ref):
    scale = lax.rsqrt(1.0 + d0_ref[...] + d1_ref[...])
    o_ref[...] = scale * (g0_ref[...] + g1_ref[...] + mx_ref[...]) + b_ref[...]


def kernel(x, features, edge_index, W, b):
    row = edge_index[0]
    col = edge_index[1]
    pad = EPAD - E
    rows_p = jnp.concatenate([row, jnp.zeros((pad,), row.dtype)]).reshape(NW, NCHUNK, CH)
    cols_p = jnp.concatenate([col, jnp.full((pad,), N, col.dtype)]).reshape(NW, NCHUNK, CH)
    xr = x.reshape(NS, BPT // CH, CH)
    feats_p = jnp.pad(features, ((0, NPAD - N), (0, 0)))

    mesh = plsc.VectorSubcoreMesh(core_axis_name="c", subcore_axis_name="s",
                                  num_cores=NC, num_subcores=NS)
    sc_params = pltpu.CompilerParams(use_tc_tiling_on_sc=False)

    deg_call = pl.kernel(
        _deg_body,
        out_type=[jax.ShapeDtypeStruct((NPAD,), jnp.float32),
                  jax.ShapeDtypeStruct((NPAD,), jnp.float32)],
        mesh=mesh,
        scratch_types=[
            pltpu.VMEM((NCHUNK, CH), jnp.int32),
            pltpu.VMEM((CH,), jnp.float32),
            pltpu.VMEM((CH,), jnp.float32),
            pltpu.VMEM_SHARED((NPAD,), jnp.float32),
        ],
        compiler_params=sc_params,
    )
    deg0, deg1 = deg_call(cols_p)

    m = pl.pallas_call(
        _mm_body,
        grid=(NPAD // RBLK,),
        in_specs=[
            pl.BlockSpec((RBLK, D), lambda i: (i, 0)),
            pl.BlockSpec((D, H), lambda i: (0, 0)),
            pl.BlockSpec((RBLK, 1), lambda i: (i, 0)),
            pl.BlockSpec((RBLK, 1), lambda i: (i, 0)),
        ],
        out_specs=pl.BlockSpec((RBLK, H), lambda i: (i, 0)),
        out_shape=jax.ShapeDtypeStruct((NPAD, H), jnp.float32),
    )(feats_p, W, deg0.reshape(NPAD, 1), deg1.reshape(NPAD, 1))

    scat_call = pl.kernel(
        _scat_body,
        out_type=[jax.ShapeDtypeStruct((B, H), jnp.float32),
                  jax.ShapeDtypeStruct((B, H), jnp.float32),
                  jax.ShapeDtypeStruct((B, H), jnp.float32),
                  jax.ShapeDtypeStruct((B,), jnp.float32),
                  jax.ShapeDtypeStruct((B,), jnp.float32)],
        mesh=mesh,
        scratch_types=[
            pltpu.VMEM((NCHUNK, CH), jnp.int32),
            pltpu.VMEM((NCHUNK, CH), jnp.int32),
            pltpu.VMEM((BPT // CH, CH), jnp.int32),
            pltpu.VMEM((4, CH, H), jnp.float32),
            pltpu.VMEM((CH, H), jnp.float32),
            pltpu.VMEM((CH,), jnp.float32),
            pltpu.VMEM_SHARED((NPAD, H), jnp.float32),
            pltpu.SemaphoreType.DMA,
            pltpu.SemaphoreType.DMA,
        ],
        compiler_params=sc_params,
    )
    g0, g1, mx, d0x, d1x = scat_call(rows_p, cols_p, xr, m, deg0, deg1)

    out = pl.pallas_call(
        _fin_body,
        out_shape=jax.ShapeDtypeStruct((B, H), jnp.float32),
    )(g0, g1, mx, d0x.reshape(B, 1), d1x.reshape(B, 1), b.reshape(1, H))
    return out


# speedup vs baseline: 59.0230x; 2.7115x over previous
import functools

import jax
import jax.numpy as jnp
from jax import lax
from jax.experimental import pallas as pl
from jax.experimental.pallas import tpu as pltpu
from jax.experimental.pallas import tpu_sc as plsc

NC, NS = 2, 16
NW = NC * NS
L = 16
CH = 128

N = 10000
NPAD = 10240
E = 320000
EPW = NPAD
NCHUNK = EPW // CH
EPAD = NW * EPW
D = 128
H = 64
B = 4096
BPT = B // NS
ROWS_PT = NPAD // NS
RBLK = 512


def _fill1d(ref, val, n):
    def st(i, _):
        ref[pl.ds(i * L, L)] = jnp.full((L,), val, jnp.float32)
        return 0
    lax.fori_loop(0, n // L, st, 0)


def _deg_body(cols_hbm, deg0_hbm, deg1_hbm, idx_v, ones_v, zbuf_v, deg_sh):
    c = lax.axis_index("c")
    sid = lax.axis_index("s")
    wid = c * NS + sid
    _fill1d(ones_v, 1.0, CH)
    _fill1d(zbuf_v, 0.0, CH)

    def zs(t, _):
        pltpu.sync_copy(zbuf_v, deg_sh.at[pl.ds(sid * ROWS_PT + t * CH, CH)])
        return 0
    lax.fori_loop(0, ROWS_PT // CH, zs, 0)
    pltpu.sync_copy(cols_hbm.at[wid], idx_v)
    plsc.subcore_barrier()

    def scat(j, _):
        pltpu.sync_copy(ones_v, deg_sh.at[idx_v.at[j]], add=True)
        return 0
    lax.fori_loop(0, NCHUNK, scat, 0)
    plsc.subcore_barrier()

    sl = pl.ds(sid * ROWS_PT, ROWS_PT)

    @pl.when(c == 0)
    def _():
        pltpu.sync_copy(deg_sh.at[sl], deg0_hbm.at[sl])

    @pl.when(c == 1)
    def _():
        pltpu.sync_copy(deg_sh.at[sl], deg1_hbm.at[sl])


def _scat_body(rows_hbm, cols_hbm, xr_hbm, m_hbm, deg0_hbm, deg1_hbm,
               g0_hbm, g1_hbm, dsum_hbm,
               ridx_v, cidx_v, x_v, buf_v, gbuf_v, dbuf_v, dbuf2_v,
               s_sh, m_sh, semg, sems):
    c = lax.axis_index("c")
    sid = lax.axis_index("s")
    wid = c * NS + sid

    def zrow(i, _):
        def zc(k, _):
            buf_v[0, i, pl.ds(k * 2 * L, 2 * L)] = jnp.zeros((2 * L,), jnp.bfloat16)
            return 0
        lax.fori_loop(0, H // (2 * L), zc, 0)
        return 0
    lax.fori_loop(0, CH, zrow, 0)

    def zs(t, _):
        pltpu.sync_copy(buf_v.at[0], s_sh.at[pl.ds(sid * ROWS_PT + t * CH, CH)])
        return 0
    lax.fori_loop(0, ROWS_PT // CH, zs, 0)

    msl = pl.ds(sid * ROWS_PT, ROWS_PT)
    pltpu.sync_copy(m_hbm.at[msl], m_sh.at[msl])

    pltpu.sync_copy(rows_hbm.at[wid], ridx_v)
    pltpu.sync_copy(cols_hbm.at[wid], cidx_v)
    pltpu.sync_copy(xr_hbm.at[sid], x_v)
    plsc.subcore_barrier()

    pltpu.async_copy(m_sh.at[ridx_v.at[0]], buf_v.at[0], semg)
    pltpu.async_copy(m_sh.at[ridx_v.at[1]], buf_v.at[1], semg)

    def ed(j, _):
        @pl.when(j >= 2)
        def _():
            pltpu.make_async_copy(buf_v.at[0], s_sh.at[cidx_v.at[0]], sems).wait()

        @pl.when(j + 2 < NCHUNK)
        def _():
            p2 = lax.rem(j + 2, 4)
            pltpu.async_copy(m_sh.at[ridx_v.at[j + 2]], buf_v.at[p2], semg)
        p = lax.rem(j, 4)
        pltpu.make_async_copy(m_sh.at[ridx_v.at[0]], buf_v.at[p], semg).wait()
        pltpu.async_copy(buf_v.at[p], s_sh.at[cidx_v.at[j]], sems, add=True)
        return 0
    lax.fori_loop(0, NCHUNK, ed, 0)
    pltpu.make_async_copy(buf_v.at[0], s_sh.at[cidx_v.at[0]], sems).wait()
    pltpu.make_async_copy(buf_v.at[1], s_sh.at[cidx_v.at[1]], sems).wait()
    plsc.subcore_barrier()

    def gk(k, _):
        idx = x_v.at[k]
        osl = pl.ds(sid * BPT + k * CH, CH)
        pltpu.sync_copy(s_sh.at[idx], gbuf_v)

        @pl.when(c == 0)
        def _():
            pltpu.sync_copy(m_sh.at[idx], buf_v.at[0])

            def addr(r, _):
                def addc(kk, _):
                    sl = pl.ds(kk * 2 * L, 2 * L)
                    gbuf_v[r, sl] = gbuf_v[r, sl] + buf_v[0, r, sl]
                    return 0
                lax.fori_loop(0, H // (2 * L), addc, 0)
                return 0
            lax.fori_loop(0, CH, addr, 0)
            pltpu.sync_copy(gbuf_v, g0_hbm.at[osl])

        @pl.when(c == 1)
        def _():
            pltpu.sync_copy(gbuf_v, g1_hbm.at[osl])
            pltpu.sync_copy(deg0_hbm.at[idx], dbuf_v)
            pltpu.sync_copy(deg1_hbm.at[idx], dbuf2_v)

            def addd(kk, _):
                sl = pl.ds(kk * L, L)
                dbuf_v[sl] = dbuf_v[sl] + dbuf2_v[sl]
                return 0
            lax.fori_loop(0, CH // L, addd, 0)
            pltpu.sync_copy(dbuf_v, dsum_hbm.at[osl])
        return 0
    lax.fori_loop(0, BPT // CH, gk, 0)


def _mm_body(f_ref, w_ref, d0_ref, d1_ref, m_ref):
    dinv = lax.rsqrt(1.0 + d0_ref[...] + d1_ref[...])
    h = jnp.dot(f_ref[...], w_ref[...], preferred_element_type=jnp.float32)
    m_ref[...] = (dinv * h).astype(jnp.bfloat16)


def _fin_body(g0_ref, g1_ref, ds_ref, b_ref, o_ref):
    scale = lax.rsqrt(1.0 + ds_ref[...])
    g = g0_ref[...].astype(jnp.float32) + g1_ref[...].astype(jnp.float32)
    o_ref[...] = scale * g + b_ref[...]


def kernel(x, features, edge_index, W, b):
    row = edge_index[0]
    col = edge_index[1]
    pad = EPAD - E
    rows_p = jnp.concatenate([row, jnp.zeros((pad,), row.dtype)]).reshape(NW, NCHUNK, CH)
    cols_p = jnp.concatenate([col, jnp.full((pad,), N, col.dtype)]).reshape(NW, NCHUNK, CH)
    xr = x.reshape(NS, BPT // CH, CH)
    feats_p = jnp.pad(features, ((0, NPAD - N), (0, 0)))

    mesh = plsc.VectorSubcoreMesh(core_axis_name="c", subcore_axis_name="s",
                                  num_cores=NC, num_subcores=NS)
    sc_params = pltpu.CompilerParams(use_tc_tiling_on_sc=False)

    deg_call = pl.kernel(
        _deg_body,
        out_type=[jax.ShapeDtypeStruct((NPAD,), jnp.float32),
                  jax.ShapeDtypeStruct((NPAD,), jnp.float32)],
        mesh=mesh,
        scratch_types=[
            pltpu.VMEM((NCHUNK, CH), jnp.int32),
            pltpu.VMEM((CH,), jnp.float32),
            pltpu.VMEM((CH,), jnp.float32),
            pltpu.VMEM_SHARED((NPAD,), jnp.float32),
        ],
        compiler_params=sc_params,
    )
    deg0, deg1 = deg_call(cols_p)

    m = pl.pallas_call(
        _mm_body,
        grid=(NPAD // RBLK,),
        in_specs=[
            pl.BlockSpec((RBLK, D), lambda i: (i, 0)),
            pl.BlockSpec((D, H), lambda i: (0, 0)),
            pl.BlockSpec((RBLK, 1), lambda i: (i, 0)),
            pl.BlockSpec((RBLK, 1), lambda i: (i, 0)),
        ],
        out_specs=pl.BlockSpec((RBLK, H), lambda i: (i, 0)),
        out_shape=jax.ShapeDtypeStruct((NPAD, H), jnp.bfloat16),
    )(feats_p, W, deg0.reshape(NPAD, 1), deg1.reshape(NPAD, 1))

    scat_call = pl.kernel(
        _scat_body,
        out_type=[jax.ShapeDtypeStruct((B, H), jnp.bfloat16),
                  jax.ShapeDtypeStruct((B, H), jnp.bfloat16),
                  jax.ShapeDtypeStruct((B,), jnp.float32)],
        mesh=mesh,
        scratch_types=[
            pltpu.VMEM((NCHUNK, CH), jnp.int32),
            pltpu.VMEM((NCHUNK, CH), jnp.int32),
            pltpu.VMEM((BPT // CH, CH), jnp.int32),
            pltpu.VMEM((4, CH, H), jnp.bfloat16),
            pltpu.VMEM((CH, H), jnp.bfloat16),
            pltpu.VMEM((CH,), jnp.float32),
            pltpu.VMEM((CH,), jnp.float32),
            pltpu.VMEM_SHARED((NPAD, H), jnp.bfloat16),
            pltpu.VMEM_SHARED((NPAD, H), jnp.bfloat16),
            pltpu.SemaphoreType.DMA,
            pltpu.SemaphoreType.DMA,
        ],
        compiler_params=sc_params,
    )
    g0, g1, dsum = scat_call(rows_p, cols_p, xr, m, deg0, deg1)

    out = pl.pallas_call(
        _fin_body,
        out_shape=jax.ShapeDtypeStruct((B, H), jnp.float32),
    )(g0, g1, dsum.reshape(B, 1), b.reshape(1, H))
    return out


# speedup vs baseline: 64.0437x; 1.0851x over previous
import functools

import jax
import jax.numpy as jnp
from jax import lax
from jax.experimental import pallas as pl
from jax.experimental.pallas import tpu as pltpu
from jax.experimental.pallas import tpu_sc as plsc

NC, NS = 2, 16
NW = NC * NS
L = 16
CH = 128

N = 10000
NPAD = 10240
E = 320000
EPW = NPAD
NCHUNK = EPW // CH
EPAD = NW * EPW
D = 128
H = 64
B = 4096
BPT = B // NS
ROWS_PT = NPAD // NS
MROWS_PT = N // NS
RBLK = 2000


def _fill1d(ref, val, n):
    def st(i, _):
        ref[pl.ds(i * L, L)] = jnp.full((L,), val, jnp.float32)
        return 0
    lax.fori_loop(0, n // L, st, 0)


def _deg_body(cols_hbm, deg0_hbm, deg1_hbm, idx_v, ones_v, zbuf_v, deg_sh, sem):
    c = lax.axis_index("c")
    sid = lax.axis_index("s")
    wid = c * NS + sid
    _fill1d(ones_v, 1.0, CH)
    _fill1d(zbuf_v, 0.0, CH)

    pltpu.async_copy(cols_hbm.at[wid], idx_v, sem)

    def zs(t, _):
        pltpu.sync_copy(zbuf_v, deg_sh.at[pl.ds(sid * ROWS_PT + t * CH, CH)])
        return 0
    lax.fori_loop(0, ROWS_PT // CH, zs, 0)
    pltpu.make_async_copy(cols_hbm.at[wid], idx_v, sem).wait()
    plsc.subcore_barrier()

    def scat(j, _):
        pltpu.async_copy(ones_v, deg_sh.at[idx_v.at[j]], sem, add=True)

        @pl.when(j >= 4)
        def _():
            pltpu.make_async_copy(ones_v, deg_sh.at[idx_v.at[0]], sem).wait()
        return 0
    lax.fori_loop(0, NCHUNK, scat, 0)

    def drain(j, _):
        pltpu.make_async_copy(ones_v, deg_sh.at[idx_v.at[0]], sem).wait()
        return 0
    lax.fori_loop(0, 4, drain, 0)
    plsc.subcore_barrier()

    sl = pl.ds(sid * ROWS_PT, ROWS_PT)

    @pl.when(c == 0)
    def _():
        pltpu.sync_copy(deg_sh.at[sl], deg0_hbm.at[sl])

    @pl.when(c == 1)
    def _():
        pltpu.sync_copy(deg_sh.at[sl], deg1_hbm.at[sl])


def _scat_body(rows_hbm, cols_hbm, xr_hbm, m_hbm, deg0_hbm, deg1_hbm,
               g0_hbm, g1_hbm, dsum_hbm,
               ridx_v, cidx_v, x_v, buf_v, gbuf_v, dbuf_v, dbuf2_v,
               s_sh, m_sh, semg, sems):
    c = lax.axis_index("c")
    sid = lax.axis_index("s")
    wid = c * NS + sid

    def zrow(i, _):
        def zc(k, _):
            buf_v[0, i, pl.ds(k * 2 * L, 2 * L)] = jnp.zeros((2 * L,), jnp.bfloat16)
            return 0
        lax.fori_loop(0, H // (2 * L), zc, 0)
        return 0
    lax.fori_loop(0, CH, zrow, 0)

    msl = pl.ds(sid * MROWS_PT, MROWS_PT)
    pltpu.async_copy(m_hbm.at[msl], m_sh.at[msl], semg)
    pltpu.async_copy(rows_hbm.at[wid], ridx_v, semg)
    pltpu.async_copy(cols_hbm.at[wid], cidx_v, semg)
    pltpu.async_copy(xr_hbm.at[sid], x_v, semg)

    def zs(t, _):
        pltpu.async_copy(buf_v.at[0], s_sh.at[pl.ds(sid * ROWS_PT + t * CH, CH)],
                         sems)

        @pl.when(t >= 3)
        def _():
            pltpu.make_async_copy(buf_v.at[0], s_sh.at[pl.ds(sid * ROWS_PT, CH)],
                                  sems).wait()
        return 0
    lax.fori_loop(0, ROWS_PT // CH, zs, 0)

    pltpu.make_async_copy(m_hbm.at[msl], m_sh.at[msl], semg).wait()
    pltpu.make_async_copy(rows_hbm.at[wid], ridx_v, semg).wait()
    pltpu.make_async_copy(cols_hbm.at[wid], cidx_v, semg).wait()
    pltpu.make_async_copy(xr_hbm.at[sid], x_v, semg).wait()

    def zdrain(t, _):
        pltpu.make_async_copy(buf_v.at[0], s_sh.at[pl.ds(sid * ROWS_PT, CH)],
                              sems).wait()
        return 0
    lax.fori_loop(0, 3, zdrain, 0)
    plsc.subcore_barrier()

    pltpu.async_copy(m_sh.at[ridx_v.at[0]], buf_v.at[0], semg)
    pltpu.async_copy(m_sh.at[ridx_v.at[1]], buf_v.at[1], semg)

    def ed(j, _):
        @pl.when(j >= 2)
        def _():
            pltpu.make_async_copy(buf_v.at[0], s_sh.at[cidx_v.at[0]], sems).wait()

        @pl.when(j + 2 < NCHUNK)
        def _():
            p2 = lax.rem(j + 2, 4)
            pltpu.async_copy(m_sh.at[ridx_v.at[j + 2]], buf_v.at[p2], semg)
        p = lax.rem(j, 4)
        pltpu.make_async_copy(m_sh.at[ridx_v.at[0]], buf_v.at[p], semg).wait()
        pltpu.async_copy(buf_v.at[p], s_sh.at[cidx_v.at[j]], sems, add=True)
        return 0
    lax.fori_loop(0, NCHUNK, ed, 0)
    pltpu.make_async_copy(buf_v.at[0], s_sh.at[cidx_v.at[0]], sems).wait()
    pltpu.make_async_copy(buf_v.at[1], s_sh.at[cidx_v.at[1]], sems).wait()
    plsc.subcore_barrier()

    NK = BPT // CH
    for k in range(NK):
        pltpu.async_copy(s_sh.at[x_v.at[k]], gbuf_v.at[k], semg)

    @pl.when(c == 0)
    def _():
        for k in range(NK):
            pltpu.async_copy(m_sh.at[x_v.at[k]], buf_v.at[k], semg)
        for k in range(NK):
            pltpu.make_async_copy(s_sh.at[x_v.at[k]], gbuf_v.at[k], semg).wait()
            pltpu.make_async_copy(m_sh.at[x_v.at[k]], buf_v.at[k], semg).wait()

        def addr(r, _):
            for k in range(NK):
                def addc(kk, _):
                    sl = pl.ds(kk * 2 * L, 2 * L)
                    gbuf_v[k, r, sl] = gbuf_v[k, r, sl] + buf_v[k, r, sl]
                    return 0
                lax.fori_loop(0, H // (2 * L), addc, 0)
            return 0
        lax.fori_loop(0, CH, addr, 0)
        for k in range(NK):
            pltpu.sync_copy(gbuf_v.at[k], g0_hbm.at[pl.ds(sid * BPT + k * CH, CH)])

    @pl.when(c == 1)
    def _():
        for k in range(NK):
            pltpu.async_copy(deg0_hbm.at[x_v.at[k]], dbuf_v.at[k], sems)
            pltpu.async_copy(deg1_hbm.at[x_v.at[k]], dbuf2_v.at[k], sems)
        for k in range(NK):
            pltpu.make_async_copy(s_sh.at[x_v.at[k]], gbuf_v.at[k], semg).wait()
            pltpu.make_async_copy(deg0_hbm.at[x_v.at[k]], dbuf_v.at[k], sems).wait()
            pltpu.make_async_copy(deg1_hbm.at[x_v.at[k]], dbuf2_v.at[k], sems).wait()

        def addd(kk, _):
            sl = pl.ds(kk * L, L)
            for k in range(NK):
                dbuf_v[k, sl] = dbuf_v[k, sl] + dbuf2_v[k, sl]
            return 0
        lax.fori_loop(0, CH // L, addd, 0)
        for k in range(NK):
            osl = pl.ds(sid * BPT + k * CH, CH)
            pltpu.sync_copy(gbuf_v.at[k], g1_hbm.at[osl])
            pltpu.sync_copy(dbuf_v.at[k], dsum_hbm.at[osl])


def _mm_body(f_ref, w_ref, d0_ref, d1_ref, m_ref):
    dinv = lax.rsqrt(1.0 + d0_ref[...] + d1_ref[...])
    h = jnp.dot(f_ref[...], w_ref[...], preferred_element_type=jnp.float32)
    m_ref[...] = (dinv * h).astype(jnp.bfloat16)


def _fin_body(g0_ref, g1_ref, ds_ref, b_ref, o_ref):
    scale = lax.rsqrt(1.0 + ds_ref[...])
    g = g0_ref[...].astype(jnp.float32) + g1_ref[...].astype(jnp.float32)
    o_ref[...] = scale * g + b_ref[...]


def kernel(x, features, edge_index, W, b):
    row = edge_index[0]
    col = edge_index[1]
    pad = EPAD - E
    rows_p = jnp.concatenate([row, jnp.zeros((pad,), row.dtype)]).reshape(NW, NCHUNK, CH)
    cols_p = jnp.concatenate([col, jnp.full((pad,), N, col.dtype)]).reshape(NW, NCHUNK, CH)
    xr = x.reshape(NS, BPT // CH, CH)

    mesh = plsc.VectorSubcoreMesh(core_axis_name="c", subcore_axis_name="s",
                                  num_cores=NC, num_subcores=NS)
    sc_params = pltpu.CompilerParams(use_tc_tiling_on_sc=False)

    deg_call = pl.kernel(
        _deg_body,
        out_type=[jax.ShapeDtypeStruct((NPAD,), jnp.float32),
                  jax.ShapeDtypeStruct((NPAD,), jnp.float32)],
        mesh=mesh,
        scratch_types=[
            pltpu.VMEM((NCHUNK, CH), jnp.int32),
            pltpu.VMEM((CH,), jnp.float32),
            pltpu.VMEM((CH,), jnp.float32),
            pltpu.VMEM_SHARED((NPAD,), jnp.float32),
            pltpu.SemaphoreType.DMA,
        ],
        compiler_params=sc_params,
    )
    deg0, deg1 = deg_call(cols_p)

    m = pl.pallas_call(
        _mm_body,
        grid=(N // RBLK,),
        in_specs=[
            pl.BlockSpec((RBLK, D), lambda i: (i, 0)),
            pl.BlockSpec((D, H), lambda i: (0, 0)),
            pl.BlockSpec((RBLK, 1), lambda i: (i, 0)),
            pl.BlockSpec((RBLK, 1), lambda i: (i, 0)),
        ],
        out_specs=pl.BlockSpec((RBLK, H), lambda i: (i, 0)),
        out_shape=jax.ShapeDtypeStruct((N, H), jnp.bfloat16),
    )(features, W, deg0[:N].reshape(N, 1), deg1[:N].reshape(N, 1))

    scat_call = pl.kernel(
        _scat_body,
        out_type=[jax.ShapeDtypeStruct((B, H), jnp.bfloat16),
                  jax.ShapeDtypeStruct((B, H), jnp.bfloat16),
                  jax.ShapeDtypeStruct((B,), jnp.float32)],
        mesh=mesh,
        scratch_types=[
            pltpu.VMEM((NCHUNK, CH), jnp.int32),
            pltpu.VMEM((NCHUNK, CH), jnp.int32),
            pltpu.VMEM((BPT // CH, CH), jnp.int32),
            pltpu.VMEM((4, CH, H), jnp.bfloat16),
            pltpu.VMEM((BPT // CH, CH, H), jnp.bfloat16),
            pltpu.VMEM((BPT // CH, CH), jnp.float32),
            pltpu.VMEM((BPT // CH, CH), jnp.float32),
            pltpu.VMEM_SHARED((NPAD, H), jnp.bfloat16),
            pltpu.VMEM_SHARED((N, H), jnp.bfloat16),
            pltpu.SemaphoreType.DMA,
            pltpu.SemaphoreType.DMA,
        ],
        compiler_params=sc_params,
    )
    g0, g1, dsum = scat_call(rows_p, cols_p, xr, m, deg0, deg1)

    out = pl.pallas_call(
        _fin_body,
        out_shape=jax.ShapeDtypeStruct((B, H), jnp.float32),
    )(g0, g1, dsum.reshape(B, 1), b.reshape(1, H))
    return out


# speedup vs baseline: 69.4528x; 1.0845x over previous
import functools

import jax
import jax.numpy as jnp
from jax import lax
from jax.experimental import pallas as pl
from jax.experimental.pallas import tpu as pltpu
from jax.experimental.pallas import tpu_sc as plsc

NC, NS = 2, 16
NW = NC * NS
L = 16
CH = 128

N = 10000
NPAD = 10240
E = 320000
EPW = NPAD
NCHUNK = EPW // CH
EPAD = NW * EPW
D = 128
H = 64
B = 4096
BPT = B // NS
ROWS_PT = NPAD // NS
MROWS_PT = N // NS
RBLK = 2000


def _fill1d(ref, val, n):
    def st(i, _):
        ref[pl.ds(i * L, L)] = jnp.full((L,), val, jnp.float32)
        return 0
    lax.fori_loop(0, n // L, st, 0)


def _deg_body(ei_hbm, deg0_hbm, deg1_hbm, idx_v, ones_v, zbuf_v, deg_sh, sem):
    c = lax.axis_index("c")
    sid = lax.axis_index("s")
    wid = c * NS + sid
    _fill1d(ones_v, 1.0, CH)
    _fill1d(zbuf_v, 0.0, CH)

    pltpu.async_copy(ei_hbm.at[1].at[wid], idx_v, sem)

    def zs(t, _):
        pltpu.sync_copy(zbuf_v, deg_sh.at[pl.ds(sid * ROWS_PT + t * CH, CH)])
        return 0
    lax.fori_loop(0, ROWS_PT // CH, zs, 0)
    pltpu.make_async_copy(ei_hbm.at[1].at[wid], idx_v, sem).wait()
    plsc.subcore_barrier()

    def scat(j, _):
        pltpu.async_copy(ones_v, deg_sh.at[idx_v.at[j]], sem, add=True)

        @pl.when(j >= 4)
        def _():
            pltpu.make_async_copy(ones_v, deg_sh.at[idx_v.at[0]], sem).wait()
        return 0
    lax.fori_loop(0, NCHUNK, scat, 0)

    def drain(j, _):
        pltpu.make_async_copy(ones_v, deg_sh.at[idx_v.at[0]], sem).wait()
        return 0
    lax.fori_loop(0, 4, drain, 0)
    plsc.subcore_barrier()

    sl = pl.ds(sid * ROWS_PT, ROWS_PT)

    @pl.when(c == 0)
    def _():
        pltpu.sync_copy(deg_sh.at[sl], deg0_hbm.at[sl])

    @pl.when(c == 1)
    def _():
        pltpu.sync_copy(deg_sh.at[sl], deg1_hbm.at[sl])


def _scat_body(ei_hbm, xr_hbm, m_hbm, deg0_hbm, deg1_hbm,
               g0_hbm, g1_hbm, dsum_hbm,
               ridx_v, cidx_v, x_v, buf_v, gbuf_v, dbuf_v, dbuf2_v,
               s_sh, m_sh, semg, sems):
    c = lax.axis_index("c")
    sid = lax.axis_index("s")
    wid = c * NS + sid

    def zrow(i, _):
        def zc(k, _):
            buf_v[0, i, pl.ds(k * 2 * L, 2 * L)] = jnp.zeros((2 * L,), jnp.bfloat16)
            return 0
        lax.fori_loop(0, H // (2 * L), zc, 0)
        return 0
    lax.fori_loop(0, CH, zrow, 0)

    msl = pl.ds(sid * MROWS_PT, MROWS_PT)
    pltpu.async_copy(m_hbm.at[msl], m_sh.at[msl], semg)
    pltpu.async_copy(ei_hbm.at[0].at[wid], ridx_v, semg)
    pltpu.async_copy(ei_hbm.at[1].at[wid], cidx_v, semg)
    pltpu.async_copy(xr_hbm.at[sid], x_v, semg)

    def zs(t, _):
        pltpu.async_copy(buf_v.at[0], s_sh.at[pl.ds(sid * ROWS_PT + t * CH, CH)],
                         sems)

        @pl.when(t >= 3)
        def _():
            pltpu.make_async_copy(buf_v.at[0], s_sh.at[pl.ds(sid * ROWS_PT, CH)],
                                  sems).wait()
        return 0
    lax.fori_loop(0, ROWS_PT // CH, zs, 0)

    pltpu.make_async_copy(m_hbm.at[msl], m_sh.at[msl], semg).wait()
    pltpu.make_async_copy(ei_hbm.at[0].at[wid], ridx_v, semg).wait()
    pltpu.make_async_copy(ei_hbm.at[1].at[wid], cidx_v, semg).wait()
    pltpu.make_async_copy(xr_hbm.at[sid], x_v, semg).wait()

    def zdrain(t, _):
        pltpu.make_async_copy(buf_v.at[0], s_sh.at[pl.ds(sid * ROWS_PT, CH)],
                              sems).wait()
        return 0
    lax.fori_loop(0, 3, zdrain, 0)
    plsc.subcore_barrier()

    pltpu.async_copy(m_sh.at[ridx_v.at[0]], buf_v.at[0], semg)
    pltpu.async_copy(m_sh.at[ridx_v.at[1]], buf_v.at[1], semg)

    def ed(j, _):
        @pl.when(j >= 2)
        def _():
            pltpu.make_async_copy(buf_v.at[0], s_sh.at[cidx_v.at[0]], sems).wait()

        @pl.when(j + 2 < NCHUNK)
        def _():
            p2 = lax.rem(j + 2, 4)
            pltpu.async_copy(m_sh.at[ridx_v.at[j + 2]], buf_v.at[p2], semg)
        p = lax.rem(j, 4)
        pltpu.make_async_copy(m_sh.at[ridx_v.at[0]], buf_v.at[p], semg).wait()
        pltpu.async_copy(buf_v.at[p], s_sh.at[cidx_v.at[j]], sems, add=True)
        return 0
    lax.fori_loop(0, NCHUNK, ed, 0)
    pltpu.make_async_copy(buf_v.at[0], s_sh.at[cidx_v.at[0]], sems).wait()
    pltpu.make_async_copy(buf_v.at[1], s_sh.at[cidx_v.at[1]], sems).wait()
    plsc.subcore_barrier()

    NK = BPT // CH
    for k in range(NK):
        pltpu.async_copy(s_sh.at[x_v.at[k]], gbuf_v.at[k], semg)

    @pl.when(c == 0)
    def _():
        for k in range(NK):
            pltpu.async_copy(m_sh.at[x_v.at[k]], buf_v.at[k], semg)
        for k in range(NK):
            pltpu.make_async_copy(s_sh.at[x_v.at[k]], gbuf_v.at[k], semg).wait()
            pltpu.make_async_copy(m_sh.at[x_v.at[k]], buf_v.at[k], semg).wait()

        def addr(r, _):
            for k in range(NK):
                def addc(kk, _):
                    sl = pl.ds(kk * 2 * L, 2 * L)
                    gbuf_v[k, r, sl] = gbuf_v[k, r, sl] + buf_v[k, r, sl]
                    return 0
                lax.fori_loop(0, H // (2 * L), addc, 0)
            return 0
        lax.fori_loop(0, CH, addr, 0)
        for k in range(NK):
            pltpu.sync_copy(gbuf_v.at[k], g0_hbm.at[pl.ds(sid * BPT + k * CH, CH)])

    @pl.when(c == 1)
    def _():
        for k in range(NK):
            pltpu.async_copy(deg0_hbm.at[x_v.at[k]], dbuf_v.at[k], sems)
            pltpu.async_copy(deg1_hbm.at[x_v.at[k]], dbuf2_v.at[k], sems)
        for k in range(NK):
            pltpu.make_async_copy(s_sh.at[x_v.at[k]], gbuf_v.at[k], semg).wait()
            pltpu.make_async_copy(deg0_hbm.at[x_v.at[k]], dbuf_v.at[k], sems).wait()
            pltpu.make_async_copy(deg1_hbm.at[x_v.at[k]], dbuf2_v.at[k], sems).wait()

        def addd(kk, _):
            sl = pl.ds(kk * L, L)
            for k in range(NK):
                dbuf_v[k, sl] = dbuf_v[k, sl] + dbuf2_v[k, sl]
            return 0
        lax.fori_loop(0, CH // L, addd, 0)
        for k in range(NK):
            osl = pl.ds(sid * BPT + k * CH, CH)
            pltpu.sync_copy(gbuf_v.at[k], g1_hbm.at[osl])
            pltpu.sync_copy(dbuf_v.at[k], dsum_hbm.at[osl])


def _mm_body(f_ref, w_ref, d0_ref, d1_ref, m_ref):
    dinv = lax.rsqrt(1.0 + d0_ref[...] + d1_ref[...])
    h = jnp.dot(f_ref[...], w_ref[...], preferred_element_type=jnp.float32)
    m_ref[...] = (dinv * h).astype(jnp.bfloat16)


def _fin_body(g0_ref, g1_ref, ds_ref, b_ref, o_ref):
    scale = lax.rsqrt(1.0 + ds_ref[...])
    g = g0_ref[...].astype(jnp.float32) + g1_ref[...].astype(jnp.float32)
    o_ref[...] = scale * g + b_ref[...]


def kernel(x, features, edge_index, W, b):
    ei_p = jnp.pad(edge_index, ((0, 0), (0, EPAD - E)),
                   constant_values=N).reshape(2, NW, NCHUNK, CH)
    xr = x.reshape(NS, BPT // CH, CH)

    mesh = plsc.VectorSubcoreMesh(core_axis_name="c", subcore_axis_name="s",
                                  num_cores=NC, num_subcores=NS)
    sc_params = pltpu.CompilerParams(use_tc_tiling_on_sc=False)

    deg_call = pl.kernel(
        _deg_body,
        out_type=[jax.ShapeDtypeStruct((NPAD,), jnp.float32),
                  jax.ShapeDtypeStruct((NPAD,), jnp.float32)],
        mesh=mesh,
        scratch_types=[
            pltpu.VMEM((NCHUNK, CH), jnp.int32),
            pltpu.VMEM((CH,), jnp.float32),
            pltpu.VMEM((CH,), jnp.float32),
            pltpu.VMEM_SHARED((NPAD,), jnp.float32),
            pltpu.SemaphoreType.DMA,
        ],
        compiler_params=sc_params,
    )
    deg0, deg1 = deg_call(ei_p)

    m = pl.pallas_call(
        _mm_body,
        grid=(N // RBLK,),
        in_specs=[
            pl.BlockSpec((RBLK, D), lambda i: (i, 0)),
            pl.BlockSpec((D, H), lambda i: (0, 0)),
            pl.BlockSpec((RBLK, 1), lambda i: (i, 0)),
            pl.BlockSpec((RBLK, 1), lambda i: (i, 0)),
        ],
        out_specs=pl.BlockSpec((RBLK, H), lambda i: (i, 0)),
        out_shape=jax.ShapeDtypeStruct((N, H), jnp.bfloat16),
    )(features, W, deg0[:N].reshape(N, 1), deg1[:N].reshape(N, 1))

    scat_call = pl.kernel(
        _scat_body,
        out_type=[jax.ShapeDtypeStruct((B, H), jnp.bfloat16),
                  jax.ShapeDtypeStruct((B, H), jnp.bfloat16),
                  jax.ShapeDtypeStruct((B,), jnp.float32)],
        mesh=mesh,
        scratch_types=[
            pltpu.VMEM((NCHUNK, CH), jnp.int32),
            pltpu.VMEM((NCHUNK, CH), jnp.int32),
            pltpu.VMEM((BPT // CH, CH), jnp.int32),
            pltpu.VMEM((4, CH, H), jnp.bfloat16),
            pltpu.VMEM((BPT // CH, CH, H), jnp.bfloat16),
            pltpu.VMEM((BPT // CH, CH), jnp.float32),
            pltpu.VMEM((BPT // CH, CH), jnp.float32),
            pltpu.VMEM_SHARED((NPAD, H), jnp.bfloat16),
            pltpu.VMEM_SHARED((NPAD, H), jnp.bfloat16),
            pltpu.SemaphoreType.DMA,
            pltpu.SemaphoreType.DMA,
        ],
        compiler_params=sc_params,
    )
    g0, g1, dsum = scat_call(ei_p, xr, m, deg0, deg1)

    out = pl.pallas_call(
        _fin_body,
        out_shape=jax.ShapeDtypeStruct((B, H), jnp.float32),
    )(g0, g1, dsum.reshape(B, 1), b.reshape(1, H))
    return out
